# baseline (speedup 1.0000x reference)
import jax
import jax.numpy as jnp
from jax.experimental import pallas as pl
from jax.experimental.pallas import tpu as pltpu

_N = 4096
_B = 4
_K = 5
_F1 = 20
_F2 = 30
_C = 10
_CH = 128
_NCH = _N // _CH

_HP = jax.lax.Precision.DEFAULT
_BF = jnp.bfloat16


def _mega_kernel(lchunk_ref, x0_ref, m1_ref, m2_ref, b1_ref, b2_ref,
                 wfc_ref, msk_ref, sb_ref, sc_ref, bfc_ref, out_ref,
                 lb, hb, sa, sb_s, out2):
    p = pl.program_id(0)

    def lmul(full_f32):
        return jax.lax.dot_general(
            lb[...], full_f32.astype(_BF),
            dimension_numbers=(((1,), (0,)), ((), ())),
            preferred_element_type=jnp.float32, precision=_HP)

    m1 = m1_ref[...]
    m2 = m2_ref[...]

    @pl.when(p < _NCH)
    def _():
        r0 = p * _CH
        lblk = lchunk_ref[...]
        lb[pl.ds(r0, _CH), :] = lblk.astype(_BF)
        t1 = jax.lax.dot(lblk, x0_ref[...], precision=_HP)
        hb[pl.ds(r0, _CH), 0:4] = t1
        x0b = x0_ref[pl.ds(r0, _CH), :]
        sa[pl.ds(r0, _CH), 0:80] = (
            jax.lax.dot(x0b, m1[0:4], precision=_HP)
            + jax.lax.dot(t1, m1[4:8], precision=_HP))

    @pl.when(p == _NCH + 0)
    def _():
        t2 = 2.0 * lmul(hb[:, 0:4]) - x0_ref[...]
        hb[:, 4:8] = hb[:, 0:4]
        hb[:, 0:4] = t2
        sa[:, 0:80] += jax.lax.dot(t2, m1[8:12], precision=_HP)

    @pl.when(p == _NCH + 1)
    def _():
        t3 = 2.0 * lmul(hb[:, 0:4]) - hb[:, 4:8]
        hb[:, 4:8] = hb[:, 0:4]
        hb[:, 0:4] = t3
        sa[:, 0:80] += jax.lax.dot(t3, m1[12:16], precision=_HP)

    @pl.when(p == _NCH + 2)
    def _():
        t4 = 2.0 * lmul(hb[:, 0:4]) - hb[:, 4:8]
        acc = sa[:, 0:80] + jax.lax.dot(t4, m1[16:20], precision=_HP)
        hb[:, 0:80] = jnp.maximum(acc + b1_ref[...], 0.0)

    @pl.when(p == _NCH + 3)
    def _():
        s1 = lmul(hb[:, 0:80])
        sa[:, 0:80] = s1
        out2[...] = (jax.lax.dot(hb[:, 0:80], m2[0:80], precision=_HP)
                        + jax.lax.dot(s1, m2[80:160], precision=_HP))

    @pl.when(p == _NCH + 4)
    def _():
        s2 = 2.0 * lmul(sa[:, 0:80]) - hb[:, 0:80]
        sb_s[:, 0:80] = s2
        out2[...] += jax.lax.dot(s2, m2[160:240], precision=_HP)

    @pl.when(p == _NCH + 5)
    def _():
        s3 = 2.0 * lmul(sb_s[:, 0:80]) - sa[:, 0:80]
        sa[:, 0:80] = s3
        out2[...] += jax.lax.dot(s3, m2[240:320], precision=_HP)

    @pl.when(p == _NCH + 6)
    def _():
        s4 = 2.0 * lmul(sa[:, 0:80]) - sb_s[:, 0:80]
        acc = out2[...] + jax.lax.dot(s4, m2[320:400], precision=_HP)
        h2 = jnp.maximum(acc + b2_ref[...], 0.0)
        u = jax.lax.dot_general(h2.astype(_BF), wfc_ref[...],
                                dimension_numbers=(((0,), (0,)), ((), ())),
                                preferred_element_type=jnp.float32,
                                precision=_HP)
        um = u * msk_ref[...]
        logits = jax.lax.dot(
            sb_ref[...], jax.lax.dot(um, sc_ref[...], precision=_HP),
            precision=_HP) + bfc_ref[...]
        m = jnp.max(logits, axis=1, keepdims=True)
        z = logits - m
        lse = jnp.log(jnp.sum(jnp.exp(z), axis=1, keepdims=True))
        out_ref[...] = z - lse


def kernel(x, L, W1, b1, W2, b2, Wfc, bfc):
    B, N = _B, _N
    X0 = x[:, :, 0].T

    eyeB = jnp.eye(B, dtype=jnp.float32)
    M1 = (W1[:, 0, :][:, None, None, :] * eyeB[None, :, :, None]
          ).reshape(_K * B, B * _F1)
    M2 = (W2[:, None, :, None, :] * eyeB[None, :, None, :, None]
          ).reshape(_K * B * _F1, B * _F2)
    b1t = jnp.tile(b1, (B,))[None, :]
    b2t = jnp.tile(b2, (B,))[None, :]

    Wfcf = Wfc.reshape(_C, N, _F2).transpose(1, 0, 2).reshape(
        N, _C * _F2).astype(jnp.bfloat16)

    r = jnp.arange(B * _F2)[:, None]
    q = jnp.arange(_C * _F2)[None, :]
    msk = ((r % _F2) == (q % _F2)).astype(jnp.float32)
    sb = (jnp.arange(B)[:, None] == (jnp.arange(B * _F2)[None, :] // _F2)
          ).astype(jnp.float32)
    sc = ((jnp.arange(_C * _F2)[:, None] // _F2) == jnp.arange(_C)[None, :]
          ).astype(jnp.float32)
    bfcr = bfc[None, :]

    out = pl.pallas_call(
        _mega_kernel,
        grid=(_NCH + 2 * _K - 3,),
        in_specs=[
            pl.BlockSpec((_CH, _N),
                         lambda p: (jnp.minimum(p, _NCH - 1), 0)),
            pl.BlockSpec((_N, _B), lambda p: (0, 0)),
            pl.BlockSpec((_K * _B, _B * _F1), lambda p: (0, 0)),
            pl.BlockSpec((_K * _B * _F1, _B * _F2), lambda p: (0, 0)),
            pl.BlockSpec((1, _B * _F1), lambda p: (0, 0)),
            pl.BlockSpec((1, _B * _F2), lambda p: (0, 0)),
            pl.BlockSpec((_N, _C * _F2), lambda p: (0, 0)),
            pl.BlockSpec((_B * _F2, _C * _F2), lambda p: (0, 0)),
            pl.BlockSpec((_B, _B * _F2), lambda p: (0, 0)),
            pl.BlockSpec((_C * _F2, _C), lambda p: (0, 0)),
            pl.BlockSpec((1, _C), lambda p: (0, 0)),
        ],
        out_specs=pl.BlockSpec((_B, _C), lambda p: (0, 0)),
        out_shape=jax.ShapeDtypeStruct((B, _C), jnp.float32),
        scratch_shapes=[
            pltpu.VMEM((_N, _N), _BF),
            pltpu.VMEM((_N, 80), jnp.float32),
            pltpu.VMEM((_N, 80), jnp.float32),
            pltpu.VMEM((_N, 80), jnp.float32),
            pltpu.VMEM((_N, _B * _F2), jnp.float32),
        ],
        compiler_params=pltpu.CompilerParams(
            dimension_semantics=("arbitrary",),
            vmem_limit_bytes=100 * 1024 * 1024,
        ),
    )(L, X0, M1, M2, b1t, b2t, Wfcf, msk, sb, sc, bfcr)
    return out


# speedup vs baseline: 11.3105x; 11.3105x over previous
import jax
import jax.numpy as jnp
from jax.experimental import pallas as pl
from jax.experimental.pallas import tpu as pltpu

_N = 4096
_B = 4
_K = 5
_F1 = 20
_F2 = 30
_C = 10
_BR = 512
_NI = _N // _BR

_HP = jax.lax.Precision.DEFAULT
_BF = jnp.bfloat16


def _mega_kernel(lhbm_ref, x0_ref, m1_ref, m2_ref, b1_ref, b2_ref,
                 wfc_ref, msk_ref, sb_ref, sc_ref, bfc_ref, out_ref,
                 lb, tbuf, hb, sa, sb_s, out2, uacc):
    p = pl.program_id(0)
    i = pl.program_id(1)
    r0 = i * _BR

    def lmul(full_f32):
        lrow = lb[pl.ds(r0, _BR), :]
        return jax.lax.dot_general(
            lrow, full_f32.astype(_BF),
            dimension_numbers=(((1,), (0,)), ((), ())),
            preferred_element_type=jnp.float32, precision=_HP)

    m1 = m1_ref[...]
    m2 = m2_ref[...]

    @pl.when(p == 0)
    def _():
        lblk = lhbm_ref[...]
        lb[pl.ds(r0, _BR), :] = lblk.astype(_BF)
        t1 = jax.lax.dot(lblk, x0_ref[...], precision=_HP)
        tbuf[pl.ds(r0, _BR), 0:4] = t1
        x0b = x0_ref[pl.ds(r0, _BR), :]
        hb[pl.ds(r0, _BR), :] = (jax.lax.dot(x0b, m1[0:4], precision=_HP)
                                 + jax.lax.dot(t1, m1[4:8], precision=_HP))

    @pl.when(p == 1)
    def _():
        t2 = (2.0 * lmul(tbuf[:, 0:4]) - x0_ref[pl.ds(r0, _BR), :])
        tbuf[pl.ds(r0, _BR), 4:8] = t2
        hb[pl.ds(r0, _BR), :] += jax.lax.dot(t2, m1[8:12], precision=_HP)

    @pl.when(p == 2)
    def _():
        t3 = 2.0 * lmul(tbuf[:, 4:8]) - tbuf[pl.ds(r0, _BR), 0:4]
        tbuf[pl.ds(r0, _BR), 8:12] = t3
        hb[pl.ds(r0, _BR), :] += jax.lax.dot(t3, m1[12:16], precision=_HP)

    @pl.when(p == 3)
    def _():
        t4 = 2.0 * lmul(tbuf[:, 8:12]) - tbuf[pl.ds(r0, _BR), 4:8]
        acc = hb[pl.ds(r0, _BR), :] + jax.lax.dot(t4, m1[16:20],
                                                  precision=_HP)
        hb[pl.ds(r0, _BR), :] = jnp.maximum(acc + b1_ref[...], 0.0)

    @pl.when(p == 4)
    def _():
        s1 = lmul(hb[...])
        sa[pl.ds(r0, _BR), :] = s1
        hblk = hb[pl.ds(r0, _BR), :]
        out2[pl.ds(r0, _BR), :] = (
            jax.lax.dot(hblk, m2[0:80], precision=_HP)
            + jax.lax.dot(s1, m2[80:160], precision=_HP))

    @pl.when(p == 5)
    def _():
        s2 = 2.0 * lmul(sa[...]) - hb[pl.ds(r0, _BR), :]
        sb_s[pl.ds(r0, _BR), :] = s2
        out2[pl.ds(r0, _BR), :] += jax.lax.dot(s2, m2[160:240], precision=_HP)

    @pl.when(p == 6)
    def _():
        s3 = 2.0 * lmul(sb_s[...]) - sa[pl.ds(r0, _BR), :]
        sa[pl.ds(r0, _BR), :] = s3
        out2[pl.ds(r0, _BR), :] += jax.lax.dot(s3, m2[240:320], precision=_HP)

    @pl.when(p == 7)
    def _():
        s4 = 2.0 * lmul(sa[...]) - sb_s[pl.ds(r0, _BR), :]
        acc = out2[pl.ds(r0, _BR), :] + jax.lax.dot(s4, m2[320:400],
                                                    precision=_HP)
        h2 = jnp.maximum(acc + b2_ref[...], 0.0)
        upart = jax.lax.dot_general(h2, wfc_ref[...],
                                    dimension_numbers=(((0,), (0,)), ((), ())),
                                    precision=_HP)

        @pl.when(i == 0)
        def _():
            uacc[...] = upart

        @pl.when(i > 0)
        def _():
            uacc[...] += upart

        @pl.when(i == _NI - 1)
        def _():
            um = uacc[...] * msk_ref[...]
            logits = jax.lax.dot(
                sb_ref[...], jax.lax.dot(um, sc_ref[...], precision=_HP),
                precision=_HP) + bfc_ref[...]
            m = jnp.max(logits, axis=1, keepdims=True)
            z = logits - m
            lse = jnp.log(jnp.sum(jnp.exp(z), axis=1, keepdims=True))
            out_ref[...] = z - lse


def kernel(x, L, W1, b1, W2, b2, Wfc, bfc):
    B, N = _B, _N
    X0 = x[:, :, 0].T

    eyeB = jnp.eye(B, dtype=jnp.float32)
    M1 = (W1[:, 0, :][:, None, None, :] * eyeB[None, :, :, None]
          ).reshape(_K * B, B * _F1)
    M2 = (W2[:, None, :, None, :] * eyeB[None, :, None, :, None]
          ).reshape(_K * B * _F1, B * _F2)
    b1t = jnp.tile(b1, (B,))[None, :]
    b2t = jnp.tile(b2, (B,))[None, :]

    Wfcf = Wfc.reshape(_C, N, _F2).transpose(1, 0, 2).reshape(N, _C * _F2)

    r = jnp.arange(B * _F2)[:, None]
    q = jnp.arange(_C * _F2)[None, :]
    msk = ((r % _F2) == (q % _F2)).astype(jnp.float32)
    sb = (jnp.arange(B)[:, None] == (jnp.arange(B * _F2)[None, :] // _F2)
          ).astype(jnp.float32)
    sc = ((jnp.arange(_C * _F2)[:, None] // _F2) == jnp.arange(_C)[None, :]
          ).astype(jnp.float32)
    bfcr = bfc[None, :]

    grid = (2 * _K - 2, _NI)
    out = pl.pallas_call(
        _mega_kernel,
        grid=grid,
        in_specs=[
            pl.BlockSpec((_BR, _N),
                         lambda p, i: (jnp.where(p == 0, i, _NI - 1), 0)),
            pl.BlockSpec((_N, _B), lambda p, i: (0, 0)),
            pl.BlockSpec((_K * _B, _B * _F1), lambda p, i: (0, 0)),
            pl.BlockSpec((_K * _B * _F1, _B * _F2), lambda p, i: (0, 0)),
            pl.BlockSpec((1, _B * _F1), lambda p, i: (0, 0)),
            pl.BlockSpec((1, _B * _F2), lambda p, i: (0, 0)),
            pl.BlockSpec((_BR, _C * _F2),
                         lambda p, i: (jnp.where(p == 7, i, 0), 0)),
            pl.BlockSpec((_B * _F2, _C * _F2), lambda p, i: (0, 0)),
            pl.BlockSpec((_B, _B * _F2), lambda p, i: (0, 0)),
            pl.BlockSpec((_C * _F2, _C), lambda p, i: (0, 0)),
            pl.BlockSpec((1, _C), lambda p, i: (0, 0)),
        ],
        out_specs=pl.BlockSpec((_B, _C), lambda p, i: (0, 0)),
        out_shape=jax.ShapeDtypeStruct((B, _C), jnp.float32),
        scratch_shapes=[
            pltpu.VMEM((_N, _N), _BF),
            pltpu.VMEM((_N, 16), jnp.float32),
            pltpu.VMEM((_N, _B * _F1), jnp.float32),
            pltpu.VMEM((_N, _B * _F1), jnp.float32),
            pltpu.VMEM((_N, _B * _F1), jnp.float32),
            pltpu.VMEM((_N, _B * _F2), jnp.float32),
            pltpu.VMEM((_B * _F2, _C * _F2), jnp.float32),
        ],
        compiler_params=pltpu.CompilerParams(
            dimension_semantics=("arbitrary", "arbitrary"),
            vmem_limit_bytes=100 * 1024 * 1024,
        ),
    )(L, X0, M1, M2, b1t, b2t, Wfcf, msk, sb, sc, bfcr)
    return out
